# baseline (speedup 1.0000x reference)
import functools

import jax
import jax.numpy as jnp
from jax import lax
from jax.experimental import pallas as pl
from jax.experimental.pallas import tpu as pltpu
from jax.experimental.pallas import tpu_sc as plsc

N = 10000
E = 320000
D = 128
NTILES = 16
KCA = 128
NCHUNKA = 160
NBA = 16
KC = 128
KH = KC // 2
NCHUNK = 160
NB = 16
NBLK = NCHUNK // NB
EPT = NCHUNK * KC
E_PAD = NTILES * EPT
NPAD = 10240
RPT = NPAD // NTILES
RB = 128
DEGW = 16
LAG = 4

_sc_mesh = plsc.VectorSubcoreMesh(core_axis_name="c", subcore_axis_name="s")


@functools.partial(
    pl.kernel,
    out_type=[jax.ShapeDtypeStruct((NPAD, DEGW), jnp.float32),
              jax.ShapeDtypeStruct((NPAD, DEGW), jnp.float32)],
    mesh=_sc_mesh,
    scratch_types=[
        pltpu.VMEM_SHARED((NPAD, DEGW), jnp.float32),
        pltpu.VMEM((RB, DEGW), jnp.float32),
        pltpu.VMEM((KCA, DEGW), jnp.float32),
        pltpu.VMEM((NBA, KCA), jnp.int32),
        pltpu.SemaphoreType.DMA,
    ],
)
def _deg_kernel(dst3_1, dst3_2, deg1_out, deg2_out, deg_sp, zbuf, ones,
                dstb, asem):
    c = lax.axis_index("c")
    s = lax.axis_index("s")

    def fillz(i, _):
        zbuf[i, :] = jnp.zeros((DEGW,), jnp.float32)
        return 0

    lax.fori_loop(0, RB, fillz, 0)

    def fill1(i, _):
        ones[i, :] = jnp.ones((DEGW,), jnp.float32)
        return 0

    lax.fori_loop(0, KCA, fill1, 0)

    def run(dst3, out_hbm):
        row0 = RPT * s
        for t in range(RPT // RB):
            pltpu.sync_copy(zbuf, deg_sp.at[pl.ds(row0 + RB * t, RB)])
        plsc.subcore_barrier()

        def blk(bb, _):
            pltpu.sync_copy(dst3.at[s, pl.ds(bb * NBA, NBA)], dstb)

            def step(j, _2):
                pltpu.sync_copy(ones, deg_sp.at[dstb.at[j]], add=True)
                return 0

            lax.fori_loop(0, NBA, step, 0)
            return 0

        lax.fori_loop(0, NCHUNKA // NBA, blk, 0)
        plsc.subcore_barrier()
        pltpu.sync_copy(deg_sp.at[pl.ds(row0, RPT)],
                        out_hbm.at[pl.ds(row0, RPT)])

    @pl.when(c == 0)
    def _():
        run(dst3_1, deg1_out)

    @pl.when(c == 1)
    def _():
        run(dst3_2, deg2_out)


@functools.partial(
    pl.kernel,
    out_type=[jax.ShapeDtypeStruct((NPAD, D), jnp.float32),
              jax.ShapeDtypeStruct((NPAD, D), jnp.float32)],
    mesh=_sc_mesh,
    scratch_types=[
        pltpu.VMEM_SHARED((NPAD, D), jnp.float32),
        pltpu.VMEM((KC, D), jnp.float32),
        pltpu.VMEM((KC, D), jnp.float32),
        pltpu.VMEM((NB, KC), jnp.int32),
        pltpu.VMEM((NB, KC), jnp.int32),
        pltpu.SemaphoreType.DMA,
        pltpu.SemaphoreType.DMA,
        pltpu.SemaphoreType.DMA,
        pltpu.SemaphoreType.DMA,
    ],
)
def _scatter_kernel(y1, src3_1, dst3_1, y2, src3_2, dst3_2, s1_out, s2_out,
                    acc_sp, rows0, rows1, srcb, dstb,
                    gsem0a, gsem0b, gsem1a, gsem1b):
    c = lax.axis_index("c")
    s = lax.axis_index("s")

    def run(y_hbm, src3, dst3, out_hbm):
        rows = (rows0, rows1)
        sems = ((gsem0a, gsem0b), (gsem1a, gsem1b))

        def gather(j, buf):
            for h in range(2):
                pltpu.async_copy(
                    y_hbm.at[srcb.at[j, pl.ds(h * KH, KH)]],
                    rows[buf].at[pl.ds(h * KH, KH)], sems[buf][h])

        def gwait(j, buf):
            for h in range(2):
                pltpu.make_async_copy(
                    y_hbm.at[srcb.at[j, pl.ds(h * KH, KH)]],
                    rows[buf].at[pl.ds(h * KH, KH)], sems[buf][h]).wait()

        def fillz(j, _):
            for k in range(D // 16):
                rows0[j, pl.ds(k * 16, 16)] = jnp.zeros((16,), jnp.float32)
            return 0

        lax.fori_loop(0, KC, fillz, 0)
        row0 = RPT * s
        for t in range(RPT // KC):
            pltpu.sync_copy(rows0, acc_sp.at[pl.ds(row0 + KC * t, KC)])
        plsc.subcore_barrier()

        def blk(bb, _):
            pltpu.sync_copy(src3.at[s, pl.ds(bb * NB, NB)], srcb)
            pltpu.sync_copy(dst3.at[s, pl.ds(bb * NB, NB)], dstb)
            gather(0, 0)

            def outer(gg, _):
                for b in range(2):
                    j = gg * 2 + b
                    gwait(j, b)

                    @pl.when(j + 1 < NB)
                    def _():
                        gather(j + 1, 1 - b)

                    pltpu.sync_copy(rows[b], acc_sp.at[dstb.at[j]],
                                    add=True)
                return 0

            lax.fori_loop(0, NB // 2, outer, 0)
            return 0

        lax.fori_loop(0, NBLK, blk, 0)
        plsc.subcore_barrier()
        for t in range(RPT // KC):
            pltpu.sync_copy(acc_sp.at[pl.ds(row0 + KC * t, KC)],
                            out_hbm.at[pl.ds(row0 + KC * t, KC)])

    @pl.when(c == 0)
    def _():
        run(y1, src3_1, dst3_1, s1_out)

    @pl.when(c == 1)
    def _():
        run(y2, src3_2, dst3_2, s2_out)


def _lin_body(x_ref, deg_ref, w_ref, y_ref, dinv_ref):
    dinv = lax.rsqrt(deg_ref[:, :] + 1.0)
    dinv_ref[:, :] = dinv
    y_ref[:, :] = jnp.dot(x_ref[:, :] * dinv, w_ref[:, :],
                          preferred_element_type=jnp.float32)


def _linear(x, deg, W):
    BR = 2000
    return pl.pallas_call(
        _lin_body,
        grid=(N // BR,),
        in_specs=[pl.BlockSpec((BR, D), lambda i: (i, 0)),
                  pl.BlockSpec((BR, 1), lambda i: (i, 0)),
                  pl.BlockSpec((D, D), lambda i: (0, 0))],
        out_specs=[pl.BlockSpec((BR, D), lambda i: (i, 0)),
                   pl.BlockSpec((BR, 1), lambda i: (i, 0))],
        out_shape=[jax.ShapeDtypeStruct((N, D), jnp.float32),
                   jax.ShapeDtypeStruct((N, 1), jnp.float32)],
    )(x, deg, W)


def _fin_body(s_ref, y_ref, dinv_ref, b_ref, o_ref):
    o_ref[:, :] = jnp.maximum(
        dinv_ref[:, :] * (s_ref[:, :] + y_ref[:, :]) + b_ref[:, :], 0.0)


def _finish(S, y, dinv, b2d):
    BR = 2000
    return pl.pallas_call(
        _fin_body,
        grid=(N // BR,),
        in_specs=[pl.BlockSpec((BR, D), lambda i: (i, 0)),
                  pl.BlockSpec((BR, D), lambda i: (i, 0)),
                  pl.BlockSpec((BR, 1), lambda i: (i, 0)),
                  pl.BlockSpec((1, D), lambda i: (0, 0))],
        out_specs=pl.BlockSpec((BR, D), lambda i: (i, 0)),
        out_shape=jax.ShapeDtypeStruct((N, D), jnp.float32),
    )(S, y, dinv, b2d)


def _pad_edges(edge_index):
    pad = E_PAD - E
    src = edge_index[0].astype(jnp.int32)
    dst = edge_index[1].astype(jnp.int32)
    srcp = jnp.concatenate([src, jnp.zeros((pad,), jnp.int32)])
    dstp = jnp.concatenate([dst, jnp.full((pad,), NPAD - 1, jnp.int32)])
    return srcp, dstp


def kernel(x1, edge_index1, x2, edge_index2, W, b):
    srcp1, dstp1 = _pad_edges(edge_index1)
    srcp2, dstp2 = _pad_edges(edge_index2)
    src3_1 = srcp1.reshape(NTILES, NCHUNK, KC)
    dst3_1 = dstp1.reshape(NTILES, NCHUNK, KC)
    src3_2 = srcp2.reshape(NTILES, NCHUNK, KC)
    dst3_2 = dstp2.reshape(NTILES, NCHUNK, KC)
    dst3a_1 = dstp1.reshape(NTILES, NCHUNKA, KCA)
    dst3a_2 = dstp2.reshape(NTILES, NCHUNKA, KCA)

    deg2d_1, deg2d_2 = _deg_kernel(dst3a_1, dst3a_2)
    deg1 = deg2d_1[:N, :1]
    deg2 = deg2d_2[:N, :1]

    y1, dinv1 = _linear(x1, deg1, W)
    y2, dinv2 = _linear(x2, deg2, W)

    S1p, S2p = _scatter_kernel(y1, src3_1, dst3_1, y2, src3_2, dst3_2)
    S1 = S1p[:N]
    S2 = S2p[:N]

    b2d = b.reshape(1, D)
    h1 = _finish(S1, y1, dinv1, b2d)
    h2 = _finish(S2, y2, dinv2, b2d)
    return (h1, h2)


# speedup vs baseline: 1.2934x; 1.2934x over previous
import functools

import jax
import jax.numpy as jnp
from jax import lax
from jax.experimental import pallas as pl
from jax.experimental.pallas import tpu as pltpu
from jax.experimental.pallas import tpu_sc as plsc

N = 10000
E = 320000
D = 128
NTILES = 16
KCA = 128
NCHUNKA = 160
NBA = 16
KC = 128
KH = KC // 2
NCHUNK = 160
NB = 32
NBLK = NCHUNK // NB
EPT = NCHUNK * KC
E_PAD = NTILES * EPT
NPAD = 10240
RPT = NPAD // NTILES
RB = 128
DEGW = 16
LAG = 4

_sc_mesh = plsc.VectorSubcoreMesh(core_axis_name="c", subcore_axis_name="s")


@functools.partial(
    pl.kernel,
    out_type=[jax.ShapeDtypeStruct((NPAD, DEGW), jnp.float32),
              jax.ShapeDtypeStruct((NPAD, DEGW), jnp.float32)],
    mesh=_sc_mesh,
    scratch_types=[
        pltpu.VMEM_SHARED((NPAD, DEGW), jnp.float32),
        pltpu.VMEM((RB, DEGW), jnp.float32),
        pltpu.VMEM((KCA, DEGW), jnp.float32),
        pltpu.VMEM((NBA, KCA), jnp.int32),
        pltpu.SemaphoreType.DMA,
    ],
)
def _deg_kernel(dst3_1, dst3_2, deg1_out, deg2_out, deg_sp, zbuf, ones,
                dstb, asem):
    c = lax.axis_index("c")
    s = lax.axis_index("s")

    def fillz(i, _):
        zbuf[i, :] = jnp.zeros((DEGW,), jnp.float32)
        return 0

    lax.fori_loop(0, RB, fillz, 0)

    def fill1(i, _):
        ones[i, :] = jnp.ones((DEGW,), jnp.float32)
        return 0

    lax.fori_loop(0, KCA, fill1, 0)

    def run(dst3, out_hbm):
        row0 = RPT * s
        for t in range(RPT // RB):
            pltpu.sync_copy(zbuf, deg_sp.at[pl.ds(row0 + RB * t, RB)])
        plsc.subcore_barrier()

        def blk(bb, _):
            pltpu.sync_copy(dst3.at[s, pl.ds(bb * NBA, NBA)], dstb)

            def step(j, _2):
                pltpu.async_copy(ones, deg_sp.at[dstb.at[j]], asem,
                                 add=True)

                @pl.when(j >= LAG)
                def _():
                    pltpu.make_async_copy(ones, deg_sp.at[dstb.at[0]],
                                          asem).wait()

                return 0

            lax.fori_loop(0, NBA, step, 0)
            for _k in range(LAG):
                pltpu.make_async_copy(ones, deg_sp.at[dstb.at[0]],
                                      asem).wait()
            return 0

        lax.fori_loop(0, NCHUNKA // NBA, blk, 0)
        plsc.subcore_barrier()
        pltpu.sync_copy(deg_sp.at[pl.ds(row0, RPT)],
                        out_hbm.at[pl.ds(row0, RPT)])

    @pl.when(c == 0)
    def _():
        run(dst3_1, deg1_out)

    @pl.when(c == 1)
    def _():
        run(dst3_2, deg2_out)


@functools.partial(
    pl.kernel,
    out_type=[jax.ShapeDtypeStruct((NPAD, D), jnp.float32),
              jax.ShapeDtypeStruct((NPAD, D), jnp.float32)],
    mesh=_sc_mesh,
    scratch_types=[
        pltpu.VMEM_SHARED((NPAD, D), jnp.float32),
        pltpu.VMEM((KC, D), jnp.float32),
        pltpu.VMEM((KC, D), jnp.float32),
        pltpu.VMEM((NB, KC), jnp.int32),
        pltpu.VMEM((NB, KC), jnp.int32),
        pltpu.SemaphoreType.DMA,
        pltpu.SemaphoreType.DMA,
        pltpu.SemaphoreType.DMA,
        pltpu.SemaphoreType.DMA,
    ],
)
def _scatter_kernel(y1, src3_1, dst3_1, y2, src3_2, dst3_2, s1_out, s2_out,
                    acc_sp, rows0, rows1, srcb, dstb,
                    gsem0a, gsem0b, gsem1a, gsem1b):
    c = lax.axis_index("c")
    s = lax.axis_index("s")

    def run(y_hbm, src3, dst3, out_hbm):
        rows = (rows0, rows1)
        sems = ((gsem0a, gsem0b), (gsem1a, gsem1b))

        def gather(j, buf):
            for h in range(2):
                pltpu.async_copy(
                    y_hbm.at[srcb.at[j, pl.ds(h * KH, KH)]],
                    rows[buf].at[pl.ds(h * KH, KH)], sems[buf][h])

        def gwait(j, buf):
            for h in range(2):
                pltpu.make_async_copy(
                    y_hbm.at[srcb.at[j, pl.ds(h * KH, KH)]],
                    rows[buf].at[pl.ds(h * KH, KH)], sems[buf][h]).wait()

        def fillz(j, _):
            for k in range(D // 16):
                rows0[j, pl.ds(k * 16, 16)] = jnp.zeros((16,), jnp.float32)
            return 0

        lax.fori_loop(0, KC, fillz, 0)
        row0 = RPT * s
        for t in range(RPT // KC):
            pltpu.sync_copy(rows0, acc_sp.at[pl.ds(row0 + KC * t, KC)])
        plsc.subcore_barrier()

        def blk(bb, _):
            pltpu.sync_copy(src3.at[s, pl.ds(bb * NB, NB)], srcb)
            pltpu.sync_copy(dst3.at[s, pl.ds(bb * NB, NB)], dstb)
            gather(0, 0)

            def outer(gg, _):
                for b in range(2):
                    j = gg * 2 + b

                    @pl.when(j + 1 < NB)
                    def _():
                        gather(j + 1, 1 - b)

                    gwait(j, b)
                    pltpu.sync_copy(rows[b], acc_sp.at[dstb.at[j]],
                                    add=True)
                return 0

            lax.fori_loop(0, NB // 2, outer, 0)
            return 0

        lax.fori_loop(0, NBLK, blk, 0)
        plsc.subcore_barrier()
        for t in range(RPT // KC):
            pltpu.sync_copy(acc_sp.at[pl.ds(row0 + KC * t, KC)],
                            out_hbm.at[pl.ds(row0 + KC * t, KC)])

    @pl.when(c == 0)
    def _():
        run(y1, src3_1, dst3_1, s1_out)

    @pl.when(c == 1)
    def _():
        run(y2, src3_2, dst3_2, s2_out)


def _lin_body(x_ref, deg_ref, w_ref, y_ref, dinv_ref):
    dinv = lax.rsqrt(deg_ref[:, :] + 1.0)
    dinv_ref[:, :] = dinv
    y_ref[:, :] = jnp.dot(x_ref[:, :] * dinv, w_ref[:, :],
                          preferred_element_type=jnp.float32)


def _linear(x, deg, W):
    BR = 2000
    return pl.pallas_call(
        _lin_body,
        grid=(N // BR,),
        in_specs=[pl.BlockSpec((BR, D), lambda i: (i, 0)),
                  pl.BlockSpec((BR, 1), lambda i: (i, 0)),
                  pl.BlockSpec((D, D), lambda i: (0, 0))],
        out_specs=[pl.BlockSpec((BR, D), lambda i: (i, 0)),
                   pl.BlockSpec((BR, 1), lambda i: (i, 0))],
        out_shape=[jax.ShapeDtypeStruct((N, D), jnp.float32),
                   jax.ShapeDtypeStruct((N, 1), jnp.float32)],
    )(x, deg, W)


def _fin_body(s_ref, y_ref, dinv_ref, b_ref, o_ref):
    o_ref[:, :] = jnp.maximum(
        dinv_ref[:, :] * (s_ref[:, :] + y_ref[:, :]) + b_ref[:, :], 0.0)


def _finish(S, y, dinv, b2d):
    BR = 2000
    return pl.pallas_call(
        _fin_body,
        grid=(N // BR,),
        in_specs=[pl.BlockSpec((BR, D), lambda i: (i, 0)),
                  pl.BlockSpec((BR, D), lambda i: (i, 0)),
                  pl.BlockSpec((BR, 1), lambda i: (i, 0)),
                  pl.BlockSpec((1, D), lambda i: (0, 0))],
        out_specs=pl.BlockSpec((BR, D), lambda i: (i, 0)),
        out_shape=jax.ShapeDtypeStruct((N, D), jnp.float32),
    )(S, y, dinv, b2d)


def _pad_edges(edge_index):
    pad = E_PAD - E
    src = edge_index[0].astype(jnp.int32)
    dst = edge_index[1].astype(jnp.int32)
    srcp = jnp.concatenate([src, jnp.zeros((pad,), jnp.int32)])
    dstp = jnp.concatenate([dst, jnp.full((pad,), NPAD - 1, jnp.int32)])
    return srcp, dstp


def kernel(x1, edge_index1, x2, edge_index2, W, b):
    srcp1, dstp1 = _pad_edges(edge_index1)
    srcp2, dstp2 = _pad_edges(edge_index2)
    src3_1 = srcp1.reshape(NTILES, NCHUNK, KC)
    dst3_1 = dstp1.reshape(NTILES, NCHUNK, KC)
    src3_2 = srcp2.reshape(NTILES, NCHUNK, KC)
    dst3_2 = dstp2.reshape(NTILES, NCHUNK, KC)
    dst3a_1 = dstp1.reshape(NTILES, NCHUNKA, KCA)
    dst3a_2 = dstp2.reshape(NTILES, NCHUNKA, KCA)

    deg2d_1, deg2d_2 = _deg_kernel(dst3a_1, dst3a_2)
    deg1 = deg2d_1[:N, :1]
    deg2 = deg2d_2[:N, :1]

    y1, dinv1 = _linear(x1, deg1, W)
    y2, dinv2 = _linear(x2, deg2, W)

    S1p, S2p = _scatter_kernel(y1, src3_1, dst3_1, y2, src3_2, dst3_2)

    b2d = b.reshape(1, D)
    h1 = _finish(S1p, y1, dinv1, b2d)
    h2 = _finish(S2p, y2, dinv2, b2d)
    return (h1, h2)
